# Initial kernel scaffold; baseline (speedup 1.0000x reference)
import jax
import jax.numpy as jnp
from jax.experimental import pallas as pl


def kernel(wifi_feat, rfid_feat, Wg1, bg1, ln_g, ln_b, Wg2, bg2, We1, be1, bn1_g, bn1_b, We2, be2, bn2_g, bn2_b, Wf1, bf1, bnf_g, bnf_b, Wf2, bf2):
    raise NotImplementedError("write your pallas kernel here")


# speedup vs baseline: 2.8106x; 2.8106x over previous
import functools
import math

import jax
import jax.numpy as jnp
from jax.experimental import pallas as pl

B = 2048
D_IN = 512
H = 1024
E = 16
NC = 20
TOPK = 4
EPS = 1e-5
BT = 256
NB = B // BT

_BN_SCALE = 1.0 / math.sqrt(1.0 + EPS)


_INV_SQRT2 = 1.0 / math.sqrt(2.0)


def _gelu(x):
    return x * 0.5 * (1.0 + jax.lax.erf(x * _INV_SQRT2))


def _gate_body(wifi_ref, rfid_ref, wg1_ref, bg1_ref, lng_ref, lnb_ref,
               wg2_ref, bg2_ref, w_ref):
    x = jnp.concatenate([wifi_ref[...], rfid_ref[...]], axis=1)
    g1 = jax.lax.dot_general(x, wg1_ref[...], (((1,), (1,)), ((), ())),
                             preferred_element_type=jnp.float32)
    g1 = g1 + bg1_ref[...]
    m = jnp.mean(g1, axis=1, keepdims=True)
    v = jnp.mean((g1 - m) ** 2, axis=1, keepdims=True)
    g1 = (g1 - m) * jax.lax.rsqrt(v + EPS) * lng_ref[...] + lnb_ref[...]
    g1 = _gelu(g1)
    logits = jax.lax.dot_general(g1, wg2_ref[...], (((1,), (1,)), ((), ())),
                                 preferred_element_type=jnp.float32)
    logits = logits + bg2_ref[...]
    logits = logits - jnp.max(logits, axis=1, keepdims=True)
    eg = jnp.exp(logits)
    gate = eg / jnp.sum(eg, axis=1, keepdims=True)

    eidx = jax.lax.broadcasted_iota(jnp.int32, (BT, E), 1)
    work = gate
    vals = []
    hots = []
    for _ in range(TOPK):
        mx = jnp.max(work, axis=1, keepdims=True)
        amx = jnp.argmax(work, axis=1).astype(jnp.int32)[:, None]
        hot = (eidx == amx)
        vals.append(mx)
        hots.append(hot)
        work = jnp.where(hot, -jnp.inf, work)
    v0 = vals[0]
    exps = [jnp.exp(v - v0) for v in vals]
    tot = exps[0]
    for ecur in exps[1:]:
        tot = tot + ecur
    w = jnp.zeros((BT, E), jnp.float32)
    for hot, ecur in zip(hots, exps):
        w = w + jnp.where(hot, ecur / tot, 0.0)
    w_ref[...] = w


def _expert_body(wifi_ref, rfid_ref, w_ref, we1_ref, be1_ref, bn1g_ref,
                 bn1b_ref, we2_ref, be2_ref, bn2g_ref, bn2b_ref, out_ref):
    e = pl.program_id(0)
    b = pl.program_id(1)
    x = jnp.concatenate([wifi_ref[...], rfid_ref[...]], axis=1)
    w1 = we1_ref[0]
    h1 = jax.lax.dot_general(x, w1, (((1,), (1,)), ((), ())),
                             preferred_element_type=jnp.float32)
    h1 = h1 + be1_ref[0]
    h1 = _gelu(h1 * _BN_SCALE * bn1g_ref[0] + bn1b_ref[0])
    w2 = we2_ref[0]
    h2 = jax.lax.dot_general(h1, w2, (((1,), (1,)), ((), ())),
                             preferred_element_type=jnp.float32)
    h2 = h2 + be2_ref[0]
    h2 = _gelu(h2 * _BN_SCALE * bn2g_ref[0] + bn2b_ref[0])
    eidx = jax.lax.broadcasted_iota(jnp.int32, (BT, E), 1)
    wcol = jnp.sum(jnp.where(eidx == e, w_ref[...], 0.0), axis=1)
    contrib = h2 * wcol[:, None]
    rows = pl.ds(b * BT, BT)

    @pl.when(e == 0)
    def _init():
        out_ref[rows, :] = contrib

    @pl.when(e > 0)
    def _acc():
        out_ref[rows, :] += contrib


def _final_body(fused_ref, wf1_ref, bf1_ref, bnfg_ref, bnfb_ref, wf2_ref,
                bf2_ref, out_ref):
    f1 = jax.lax.dot_general(fused_ref[...], wf1_ref[...],
                             (((1,), (1,)), ((), ())),
                             preferred_element_type=jnp.float32)
    f1 = f1 + bf1_ref[...]
    f1 = _gelu(f1 * _BN_SCALE * bnfg_ref[...] + bnfb_ref[...])
    out = jax.lax.dot_general(f1, wf2_ref[...], (((1,), (1,)), ((), ())),
                              preferred_element_type=jnp.float32)
    out_ref[...] = out + bf2_ref[...]


def kernel(wifi_feat, rfid_feat, Wg1, bg1, ln_g, ln_b, Wg2, bg2,
           We1, be1, bn1_g, bn1_b, We2, be2, bn2_g, bn2_b,
           Wf1, bf1, bnf_g, bnf_b, Wf2, bf2):
    bg1r = bg1.reshape(1, H)
    lngr = ln_g.reshape(1, H)
    lnbr = ln_b.reshape(1, H)
    bg2r = bg2.reshape(1, E)
    bf1r = bf1.reshape(1, 512)
    bnfgr = bnf_g.reshape(1, 512)
    bnfbr = bnf_b.reshape(1, 512)
    bf2r = bf2.reshape(1, NC)

    w = pl.pallas_call(
        _gate_body,
        grid=(NB,),
        in_specs=[
            pl.BlockSpec((BT, 256), lambda b: (b, 0)),
            pl.BlockSpec((BT, 256), lambda b: (b, 0)),
            pl.BlockSpec((H, D_IN), lambda b: (0, 0)),
            pl.BlockSpec((1, H), lambda b: (0, 0)),
            pl.BlockSpec((1, H), lambda b: (0, 0)),
            pl.BlockSpec((1, H), lambda b: (0, 0)),
            pl.BlockSpec((E, H), lambda b: (0, 0)),
            pl.BlockSpec((1, E), lambda b: (0, 0)),
        ],
        out_specs=pl.BlockSpec((BT, E), lambda b: (b, 0)),
        out_shape=jax.ShapeDtypeStruct((B, E), jnp.float32),
    )(wifi_feat, rfid_feat, Wg1, bg1r, lngr, lnbr, Wg2, bg2r)

    fused = pl.pallas_call(
        _expert_body,
        grid=(E, NB),
        in_specs=[
            pl.BlockSpec((BT, 256), lambda e, b: (b, 0)),
            pl.BlockSpec((BT, 256), lambda e, b: (b, 0)),
            pl.BlockSpec((BT, E), lambda e, b: (b, 0)),
            pl.BlockSpec((1, H, D_IN), lambda e, b: (e, 0, 0)),
            pl.BlockSpec((1, 1, H), lambda e, b: (e, 0, 0)),
            pl.BlockSpec((1, 1, H), lambda e, b: (e, 0, 0)),
            pl.BlockSpec((1, 1, H), lambda e, b: (e, 0, 0)),
            pl.BlockSpec((1, H, H), lambda e, b: (e, 0, 0)),
            pl.BlockSpec((1, 1, H), lambda e, b: (e, 0, 0)),
            pl.BlockSpec((1, 1, H), lambda e, b: (e, 0, 0)),
            pl.BlockSpec((1, 1, H), lambda e, b: (e, 0, 0)),
        ],
        out_specs=pl.BlockSpec((B, H), lambda e, b: (0, 0)),
        out_shape=jax.ShapeDtypeStruct((B, H), jnp.float32),
    )(wifi_feat, rfid_feat, w, We1, be1.reshape(E, 1, H),
      bn1_g.reshape(E, 1, H), bn1_b.reshape(E, 1, H), We2,
      be2.reshape(E, 1, H), bn2_g.reshape(E, 1, H), bn2_b.reshape(E, 1, H))

    out = pl.pallas_call(
        _final_body,
        grid=(NB,),
        in_specs=[
            pl.BlockSpec((BT, H), lambda b: (b, 0)),
            pl.BlockSpec((512, H), lambda b: (0, 0)),
            pl.BlockSpec((1, 512), lambda b: (0, 0)),
            pl.BlockSpec((1, 512), lambda b: (0, 0)),
            pl.BlockSpec((1, 512), lambda b: (0, 0)),
            pl.BlockSpec((NC, 512), lambda b: (0, 0)),
            pl.BlockSpec((1, NC), lambda b: (0, 0)),
        ],
        out_specs=pl.BlockSpec((BT, NC), lambda b: (b, 0)),
        out_shape=jax.ShapeDtypeStruct((B, NC), jnp.float32),
    )(fused, Wf1, bf1r, bnfgr, bnfbr, Wf2, bf2r)
    return out
